# baseline (speedup 1.0000x reference)
import functools

import jax
import jax.numpy as jnp
from jax import lax
from jax.experimental import pallas as pl
from jax.experimental.pallas import tpu as pltpu
from jax.experimental.pallas import tpu_sc as plsc

_NUM_REL = 1000
_D = 128
_K = 32
_B = 4096
_NC = 2
_NS = 16
_NW = _NC * _NS
_BPW = _B // _NW
_CB = 4
_NCHUNK = _BPW // _CB
_LANES = 16
_NSUB = _D // _LANES
_PK = plsc.PackFormat.INTERLEAVED


def _sc_body(ent_hbm, rel_hbm, node_idx_hbm, nb_idx_hbm, y_hbm,
             node_idx_v, nb_idx_v, node_rows_v, nb_buf0, nb_buf1, y_v,
             sem_node, sem0, sem1):
    c = lax.axis_index("c")
    s = lax.axis_index("s")
    w = s * _NC + c
    base = w * _BPW

    bufs = ((nb_buf0, sem0), (nb_buf1, sem1))

    for n in range(3):
        pltpu.sync_copy(node_idx_hbm.at[n, w, 0], node_idx_v)
        pltpu.sync_copy(nb_idx_hbm.at[n, w], nb_idx_v)
        table = rel_hbm if n == 1 else ent_hbm
        node_cp = pltpu.async_copy(table.at[node_idx_v], node_rows_v, sem_node)
        pltpu.async_copy(ent_hbm.at[nb_idx_v.at[0]], nb_buf0, sem0)
        pltpu.async_copy(ent_hbm.at[nb_idx_v.at[1]], nb_buf1, sem1)
        node_cp.wait()

        def half_iter(i, _):
            for b, (buf, sem) in enumerate(bufs):
                j = 2 * i + b
                pltpu.make_async_copy(
                    ent_hbm.at[nb_idx_v.at[j]], buf, sem).wait()
                for t in range(_CB):
                    lb = j * _CB + t
                    nc = [node_rows_v[lb, pl.ds(_LANES * ci, _LANES)]
                          for ci in range(_NSUB)]
                    ncb = [plsc.pack(nc[2 * ci], nc[2 * ci + 1], format=_PK)
                           for ci in range(_NSUB // 2)]
                    zero = jnp.zeros((2 * _LANES,), jnp.bfloat16)

                    @plsc.parallel_loop(0, _K, unroll=4,
                                        carry=(zero,) * (_NSUB // 2))
                    def att_acc(k, acc, t=t, ncb=ncb, buf=buf):
                        row = t * _K + k
                        vb = [buf[row, pl.ds(_LANES * ci, _LANES)]
                              for ci in range(_NSUB)]
                        vbb = [plsc.pack(vb[2 * ci], vb[2 * ci + 1],
                                         format=_PK)
                               for ci in range(_NSUB // 2)]
                        prod = [vbb[ci] * ncb[ci] for ci in range(_NSUB // 2)]
                        p = (prod[0] + prod[1]) + (prod[2] + prod[3])
                        pa, pb = plsc.unpack(p, format=_PK)
                        att = jnp.sum(pa + pb) * (1.0 / 15.0)
                        attv = jnp.full((_LANES,), att, jnp.float32)
                        attb = plsc.pack(attv, attv, format=_PK)
                        return tuple(acc[ci] + attb * vbb[ci]
                                     for ci in range(_NSUB // 2))

                    for ci in range(_NSUB // 2):
                        oa, ob = plsc.unpack(att_acc[ci], format=_PK)
                        y_v[lb, pl.ds(_LANES * 2 * ci, _LANES)] = (
                            nc[2 * ci] + oa)
                        y_v[lb, pl.ds(_LANES * (2 * ci + 1), _LANES)] = (
                            nc[2 * ci + 1] + ob)
                nxt = j + 2

                @pl.when(nxt < _NCHUNK)
                def _(buf=buf, sem=sem, nxt=nxt):
                    pltpu.async_copy(ent_hbm.at[nb_idx_v.at[nxt]], buf, sem)
            return 0

        lax.fori_loop(0, _NCHUNK // 2, half_iter, 0)
        pltpu.sync_copy(y_v, y_hbm.at[n, pl.ds(base, _BPW)])


@functools.partial(jax.jit, static_argnames=())
def _sc_attention(ent_table, rel_table, node_idx, nb_idx):
    mesh = plsc.VectorSubcoreMesh(core_axis_name="c", subcore_axis_name="s")
    f = pl.kernel(
        _sc_body,
        out_type=jax.ShapeDtypeStruct((3, _B, _D), jnp.float32),
        mesh=mesh,
        compiler_params=pltpu.CompilerParams(needs_layout_passes=False),
        scratch_types=[
            pltpu.VMEM((_BPW,), jnp.int32),
            pltpu.VMEM((_NCHUNK, _CB * _K), jnp.int32),
            pltpu.VMEM((_BPW, _D), jnp.float32),
            pltpu.VMEM((_CB * _K, _D), jnp.float32),
            pltpu.VMEM((_CB * _K, _D), jnp.float32),
            pltpu.VMEM((_BPW, _D), jnp.float32),
            pltpu.SemaphoreType.DMA,
            pltpu.SemaphoreType.DMA,
            pltpu.SemaphoreType.DMA,
        ],
    )
    return f(ent_table, rel_table, node_idx, nb_idx)


def _ln_body(y_ref, g_ref, b_ref, o_ref):
    x = y_ref[...]
    mu = jnp.mean(x, axis=-1, keepdims=True)
    xc = x - mu
    var = jnp.mean(xc * xc, axis=-1, keepdims=True)
    xn = xc * lax.rsqrt(var + 1e-5) * g_ref[...] + b_ref[...]
    o_ref[...] = jnp.transpose(xn, (1, 0, 2))


def _layer_norm_tc(y, gamma, beta):
    blk = 512
    return pl.pallas_call(
        _ln_body,
        grid=(_B // blk,),
        in_specs=[
            pl.BlockSpec((3, blk, _D), lambda i: (0, i, 0)),
            pl.BlockSpec((1, 1, _D), lambda i: (0, 0, 0)),
            pl.BlockSpec((1, 1, _D), lambda i: (0, 0, 0)),
        ],
        out_specs=pl.BlockSpec((blk, 3, _D), lambda i: (i, 0, 0)),
        out_shape=jax.ShapeDtypeStruct((_B, 3, _D), jnp.float32),
    )(y, gamma.reshape(1, 1, _D), beta.reshape(1, 1, _D))


def kernel(ent_table, rel_table, ln_gamma, ln_beta, hrts, neighbor_ids):
    hrts = hrts.astype(jnp.int32)
    nids = neighbor_ids.astype(jnp.int32)
    node_idx = jnp.stack(
        [hrts[:, 0], hrts[:, 1] % _NUM_REL, hrts[:, 2]], axis=0)
    node_idx = node_idx.reshape(3, _NW, 1, _BPW)
    nb_idx = jnp.transpose(nids, (1, 0, 2)).reshape(3, _NW, _NCHUNK, _CB * _K)

    y = _sc_attention(ent_table, rel_table, node_idx, nb_idx)
    return _layer_norm_tc(y, ln_gamma, ln_beta)


# speedup vs baseline: 1.0535x; 1.0535x over previous
import functools

import jax
import jax.numpy as jnp
from jax import lax
from jax.experimental import pallas as pl
from jax.experimental.pallas import tpu as pltpu
from jax.experimental.pallas import tpu_sc as plsc

_NUM_REL = 1000
_D = 128
_K = 32
_B = 4096
_NC = 2
_NS = 16
_NW = _NC * _NS
_BPW = _B // _NW
_CB = 4
_NCHUNK = _BPW // _CB
_LANES = 16
_NSUB = _D // _LANES
_PK = plsc.PackFormat.INTERLEAVED


def _sc_body(ent_hbm, rel_hbm, node_idx_hbm, nb_idx_hbm, y_hbm,
             node_idx_v, nb_idx_v, node_rows_v, nb_buf0, nb_buf1, y_v,
             sem_node, sem0, sem1):
    c = lax.axis_index("c")
    s = lax.axis_index("s")
    w = s * _NC + c
    base = w * _BPW

    bufs = ((nb_buf0, sem0), (nb_buf1, sem1))

    for n in range(3):
        pltpu.sync_copy(node_idx_hbm.at[n, w, 0], node_idx_v)
        pltpu.sync_copy(nb_idx_hbm.at[n, w], nb_idx_v)
        table = rel_hbm if n == 1 else ent_hbm
        node_cp = pltpu.async_copy(table.at[node_idx_v], node_rows_v, sem_node)
        pltpu.async_copy(ent_hbm.at[nb_idx_v.at[0]], nb_buf0, sem0)
        pltpu.async_copy(ent_hbm.at[nb_idx_v.at[1]], nb_buf1, sem1)
        node_cp.wait()

        def half_iter(i, _):
            for b, (buf, sem) in enumerate(bufs):
                j = 2 * i + b
                pltpu.make_async_copy(
                    ent_hbm.at[nb_idx_v.at[j]], buf, sem).wait()
                for t in range(_CB):
                    lb = j * _CB + t
                    nc = [node_rows_v[lb, pl.ds(_LANES * ci, _LANES)]
                          for ci in range(_NSUB)]
                    ncb = [plsc.pack(nc[2 * ci], nc[2 * ci + 1], format=_PK)
                           for ci in range(_NSUB // 2)]
                    zero = jnp.zeros((2 * _LANES,), jnp.bfloat16)

                    @plsc.parallel_loop(0, _K, unroll=4,
                                        carry=(zero,) * (_NSUB // 2))
                    def att_acc(k, acc, t=t, ncb=ncb, buf=buf):
                        row = t * _K + k
                        vb = [buf[row, pl.ds(_LANES * ci, _LANES)]
                              for ci in range(_NSUB)]
                        vbb = [plsc.pack(vb[2 * ci], vb[2 * ci + 1],
                                         format=_PK)
                               for ci in range(_NSUB // 2)]
                        prod = [vbb[ci] * ncb[ci] for ci in range(_NSUB // 2)]
                        p = (prod[0] + prod[1]) + (prod[2] + prod[3])
                        pa, pb = plsc.unpack(p, format=_PK)
                        att = jnp.sum(pa + pb) * (1.0 / 15.0)
                        attv = jnp.full((_LANES,), att, jnp.float32)
                        attb = plsc.pack(attv, attv, format=_PK)
                        return tuple(acc[ci] + attb * vbb[ci]
                                     for ci in range(_NSUB // 2))

                    for ci in range(_NSUB // 2):
                        oa, ob = plsc.unpack(att_acc[ci], format=_PK)
                        y_v[lb, pl.ds(_LANES * 2 * ci, _LANES)] = (
                            nc[2 * ci] + oa)
                        y_v[lb, pl.ds(_LANES * (2 * ci + 1), _LANES)] = (
                            nc[2 * ci + 1] + ob)
                nxt = j + 2

                @pl.when(nxt < _NCHUNK)
                def _(buf=buf, sem=sem, nxt=nxt):
                    pltpu.async_copy(ent_hbm.at[nb_idx_v.at[nxt]], buf, sem)
            return 0

        lax.fori_loop(0, _NCHUNK // 2, half_iter, 0)
        pltpu.sync_copy(y_v, y_hbm.at[n, pl.ds(base, _BPW)])


@functools.partial(jax.jit, static_argnames=())
def _sc_attention(ent_table, rel_table, node_idx, nb_idx):
    mesh = plsc.VectorSubcoreMesh(core_axis_name="c", subcore_axis_name="s")
    f = pl.kernel(
        _sc_body,
        out_type=jax.ShapeDtypeStruct((3, _B, _D), jnp.float32),
        mesh=mesh,
        compiler_params=pltpu.CompilerParams(needs_layout_passes=False),
        scratch_types=[
            pltpu.VMEM((_BPW,), jnp.int32),
            pltpu.VMEM((_NCHUNK, _CB * _K), jnp.int32),
            pltpu.VMEM((_BPW, _D), jnp.float32),
            pltpu.VMEM((_CB * _K, _D), jnp.float32),
            pltpu.VMEM((_CB * _K, _D), jnp.float32),
            pltpu.VMEM((_BPW, _D), jnp.float32),
            pltpu.SemaphoreType.DMA,
            pltpu.SemaphoreType.DMA,
            pltpu.SemaphoreType.DMA,
        ],
    )
    return f(ent_table, rel_table, node_idx, nb_idx)


def _ln_body(y_ref, g_ref, b_ref, o_ref):
    x = y_ref[...]
    mu = jnp.mean(x, axis=-1, keepdims=True)
    xc = x - mu
    var = jnp.mean(xc * xc, axis=-1, keepdims=True)
    o_ref[...] = xc * lax.rsqrt(var + 1e-5) * g_ref[...] + b_ref[...]


def _layer_norm_tc(y, gamma, beta):
    blk = 1024
    return pl.pallas_call(
        _ln_body,
        grid=(_B // blk,),
        in_specs=[
            pl.BlockSpec((3, blk, _D), lambda i: (0, i, 0)),
            pl.BlockSpec((1, 1, _D), lambda i: (0, 0, 0)),
            pl.BlockSpec((1, 1, _D), lambda i: (0, 0, 0)),
        ],
        out_specs=pl.BlockSpec((3, blk, _D), lambda i: (0, i, 0)),
        out_shape=jax.ShapeDtypeStruct((3, _B, _D), jnp.float32),
    )(y, gamma.reshape(1, 1, _D), beta.reshape(1, 1, _D))


def kernel(ent_table, rel_table, ln_gamma, ln_beta, hrts, neighbor_ids):
    hrts = hrts.astype(jnp.int32)
    nids = neighbor_ids.astype(jnp.int32)
    node_idx = jnp.stack(
        [hrts[:, 0], hrts[:, 1] % _NUM_REL, hrts[:, 2]], axis=0)
    node_idx = node_idx.reshape(3, _NW, 1, _BPW)
    nb_idx = jnp.transpose(nids, (1, 0, 2)).reshape(3, _NW, _NCHUNK, _CB * _K)

    y = _sc_attention(ent_table, rel_table, node_idx, nb_idx)
    out = _layer_norm_tc(y, ln_gamma, ln_beta)
    return jnp.transpose(out, (1, 0, 2))


# speedup vs baseline: 1.0843x; 1.0292x over previous
import functools

import jax
import jax.numpy as jnp
from jax import lax
from jax.experimental import pallas as pl
from jax.experimental.pallas import tpu as pltpu
from jax.experimental.pallas import tpu_sc as plsc

_NUM_REL = 1000
_D = 128
_K = 32
_B = 4096
_NC = 2
_NS = 16
_NW = _NC * _NS
_BPW = _B // _NW
_CB = 4
_NCHUNK = _BPW // _CB
_LANES = 16
_NSUB = _D // _LANES
_PK = plsc.PackFormat.INTERLEAVED


def _sc_body(ent_hbm, rel_hbm, node_idx_hbm, nb_idx_hbm, y_hbm,
             node_idx_v, nb_idx_v, node_rows_v, nb_buf0, nb_buf1, y_v,
             sem_node, sem0, sem1, sem_y):
    c = lax.axis_index("c")
    s = lax.axis_index("s")
    w = s * _NC + c
    base = w * _BPW

    bufs = ((nb_buf0, sem0), (nb_buf1, sem1))
    tables = (ent_hbm, rel_hbm, ent_hbm)

    pltpu.sync_copy(node_idx_hbm.at[0, w, 0], node_idx_v.at[0])
    pltpu.sync_copy(nb_idx_hbm.at[0, w], nb_idx_v.at[0])
    pltpu.async_copy(tables[0].at[node_idx_v.at[0]], node_rows_v.at[0],
                     sem_node)

    for n in range(3):
        cur, nxt = n % 2, (n + 1) % 2
        pltpu.async_copy(ent_hbm.at[nb_idx_v.at[cur, 0]], nb_buf0, sem0)
        pltpu.async_copy(ent_hbm.at[nb_idx_v.at[cur, 1]], nb_buf1, sem1)
        pltpu.make_async_copy(tables[n].at[node_idx_v.at[cur]],
                              node_rows_v.at[cur], sem_node).wait()
        if n < 2:
            pltpu.sync_copy(node_idx_hbm.at[n + 1, w, 0], node_idx_v.at[nxt])
            pltpu.sync_copy(nb_idx_hbm.at[n + 1, w], nb_idx_v.at[nxt])
            pltpu.async_copy(tables[n + 1].at[node_idx_v.at[nxt]],
                             node_rows_v.at[nxt], sem_node)
        if n > 0:
            pltpu.make_async_copy(
                y_v, y_hbm.at[n - 1, pl.ds(base, _BPW)], sem_y).wait()

        def half_iter(i, _, cur=cur):
            for b, (buf, sem) in enumerate(bufs):
                j = 2 * i + b
                pltpu.make_async_copy(
                    ent_hbm.at[nb_idx_v.at[cur, j]], buf, sem).wait()
                for t in range(_CB):
                    lb = j * _CB + t
                    nc = [node_rows_v[cur, lb, pl.ds(_LANES * ci, _LANES)]
                          for ci in range(_NSUB)]
                    ncb = [plsc.pack(nc[2 * ci], nc[2 * ci + 1], format=_PK)
                           for ci in range(_NSUB // 2)]
                    zero = jnp.zeros((2 * _LANES,), jnp.bfloat16)

                    @plsc.parallel_loop(0, _K, unroll=4,
                                        carry=(zero,) * (_NSUB // 2))
                    def att_acc(k, acc, t=t, ncb=ncb, buf=buf):
                        row = t * _K + k
                        vb = [buf[row, pl.ds(_LANES * ci, _LANES)]
                              for ci in range(_NSUB)]
                        vbb = [plsc.pack(vb[2 * ci], vb[2 * ci + 1],
                                         format=_PK)
                               for ci in range(_NSUB // 2)]
                        prod = [vbb[ci] * ncb[ci] for ci in range(_NSUB // 2)]
                        p = (prod[0] + prod[1]) + (prod[2] + prod[3])
                        pa, pb = plsc.unpack(p, format=_PK)
                        att = jnp.sum(pa + pb) * (1.0 / 15.0)
                        attv = jnp.full((_LANES,), att, jnp.float32)
                        attb = plsc.pack(attv, attv, format=_PK)
                        return tuple(acc[ci] + attb * vbb[ci]
                                     for ci in range(_NSUB // 2))

                    for ci in range(_NSUB // 2):
                        oa, ob = plsc.unpack(att_acc[ci], format=_PK)
                        y_v[lb, pl.ds(_LANES * 2 * ci, _LANES)] = (
                            nc[2 * ci] + oa)
                        y_v[lb, pl.ds(_LANES * (2 * ci + 1), _LANES)] = (
                            nc[2 * ci + 1] + ob)
                jn = j + 2

                @pl.when(jn < _NCHUNK)
                def _(buf=buf, sem=sem, jn=jn, cur=cur):
                    pltpu.async_copy(
                        ent_hbm.at[nb_idx_v.at[cur, jn]], buf, sem)
            return 0

        lax.fori_loop(0, _NCHUNK // 2, half_iter, 0)
        pltpu.async_copy(y_v, y_hbm.at[n, pl.ds(base, _BPW)], sem_y)

    pltpu.make_async_copy(y_v, y_hbm.at[2, pl.ds(base, _BPW)], sem_y).wait()


@functools.partial(jax.jit, static_argnames=())
def _sc_attention(ent_table, rel_table, node_idx, nb_idx):
    mesh = plsc.VectorSubcoreMesh(core_axis_name="c", subcore_axis_name="s")
    f = pl.kernel(
        _sc_body,
        out_type=jax.ShapeDtypeStruct((3, _B, _D), jnp.float32),
        mesh=mesh,
        compiler_params=pltpu.CompilerParams(needs_layout_passes=False),
        scratch_types=[
            pltpu.VMEM((2, _BPW), jnp.int32),
            pltpu.VMEM((2, _NCHUNK, _CB * _K), jnp.int32),
            pltpu.VMEM((2, _BPW, _D), jnp.float32),
            pltpu.VMEM((_CB * _K, _D), jnp.float32),
            pltpu.VMEM((_CB * _K, _D), jnp.float32),
            pltpu.VMEM((_BPW, _D), jnp.float32),
            pltpu.SemaphoreType.DMA,
            pltpu.SemaphoreType.DMA,
            pltpu.SemaphoreType.DMA,
            pltpu.SemaphoreType.DMA,
        ],
    )
    return f(ent_table, rel_table, node_idx, nb_idx)


def _ln_body(y_ref, g_ref, b_ref, o_ref):
    x = y_ref[...]
    mu = jnp.mean(x, axis=-1, keepdims=True)
    xc = x - mu
    var = jnp.mean(xc * xc, axis=-1, keepdims=True)
    o_ref[...] = xc * lax.rsqrt(var + 1e-5) * g_ref[...] + b_ref[...]


def _layer_norm_tc(y, gamma, beta):
    blk = 1024
    return pl.pallas_call(
        _ln_body,
        grid=(_B // blk,),
        in_specs=[
            pl.BlockSpec((3, blk, _D), lambda i: (0, i, 0)),
            pl.BlockSpec((1, 1, _D), lambda i: (0, 0, 0)),
            pl.BlockSpec((1, 1, _D), lambda i: (0, 0, 0)),
        ],
        out_specs=pl.BlockSpec((3, blk, _D), lambda i: (0, i, 0)),
        out_shape=jax.ShapeDtypeStruct((3, _B, _D), jnp.float32),
    )(y, gamma.reshape(1, 1, _D), beta.reshape(1, 1, _D))


def kernel(ent_table, rel_table, ln_gamma, ln_beta, hrts, neighbor_ids):
    hrts = hrts.astype(jnp.int32)
    nids = neighbor_ids.astype(jnp.int32)
    node_idx = jnp.stack(
        [hrts[:, 0], hrts[:, 1] % _NUM_REL, hrts[:, 2]], axis=0)
    node_idx = node_idx.reshape(3, _NW, 1, _BPW)
    nb_idx = jnp.transpose(nids, (1, 0, 2)).reshape(3, _NW, _NCHUNK, _CB * _K)

    y = _sc_attention(ent_table, rel_table, node_idx, nb_idx)
    out = _layer_norm_tc(y, ln_gamma, ln_beta)
    return jnp.transpose(out, (1, 0, 2))
